# baseline (speedup 1.0000x reference)
import functools
import jax
import jax.numpy as jnp
from jax import lax
from jax.experimental import pallas as pl
from jax.experimental.pallas import tpu as pltpu
from jax.experimental.pallas import tpu_sc as plsc

_NC = 2
_NS = 16
_NW = _NC * _NS
_L = 16
_UNROLL = 8


def _sc_pool(x_flat, emb_flat, B, L, D, V):
    spw = B // _NW
    groups = spw // _L
    inv_l = 1.0 / float(L)

    mesh = plsc.VectorSubcoreMesh(
        core_axis_name="c", subcore_axis_name="s",
        num_cores=_NC, num_subcores=_NS,
    )

    @functools.partial(
        pl.kernel,
        out_type=jax.ShapeDtypeStruct((B * D,), jnp.float32),
        mesh=mesh,
        compiler_params=pltpu.CompilerParams(needs_layout_passes=False),
        scratch_types=[
            pltpu.VMEM((spw * L,), jnp.int32),
            pltpu.VMEM((V * D,), jnp.float32),
            pltpu.VMEM((spw * D,), jnp.float32),
        ],
    )
    def pool_kernel(x_hbm, emb_hbm, out_hbm, x_v, emb_v, h_v):
        wid = lax.axis_index("s") * _NC + lax.axis_index("c")
        pltpu.sync_copy(x_hbm.at[pl.ds(wid * spw * L, spw * L)], x_v)
        pltpu.sync_copy(emb_hbm, emb_v)

        lane = lax.iota(jnp.int32, _L)
        row_off = lane * L
        lane_d = lane * D

        def group_body(g, carry):
            x_base = g * (_L * L)

            def tok_body(t, accs):
                l0 = t * _UNROLL
                for u in range(_UNROLL):
                    idxv = plsc.load_gather(x_v, [row_off + (x_base + l0 + u)])
                    idxd = idxv * D
                    accs = tuple(
                        accs[d] + plsc.load_gather(emb_v, [idxd + d])
                        for d in range(D)
                    )
                return accs

            zeros = jnp.zeros((_L,), jnp.float32)
            accs = lax.fori_loop(0, L // _UNROLL, tok_body, (zeros,) * D)
            h_base = g * (_L * D)
            for d in range(D):
                plsc.store_scatter(h_v, [lane_d + (h_base + d)], accs[d] * inv_l)
            return carry

        lax.fori_loop(0, groups, group_body, 0)
        pltpu.sync_copy(h_v, out_hbm.at[pl.ds(wid * spw * D, spw * D)])

    return pool_kernel(x_flat, emb_flat)


def _mlp_body(h_ref, w1_ref, b1_ref, w2_ref, b2_ref, o_ref):
    h = h_ref[...]
    z = jnp.dot(h, w1_ref[...], preferred_element_type=jnp.float32) + b1_ref[...]
    z = jnp.maximum(z, 0.0)
    o_ref[...] = jnp.dot(z, w2_ref[...], preferred_element_type=jnp.float32) + b2_ref[...]


def kernel(x, embed, W1, b1, W2, b2):
    B, L = x.shape
    V, D = embed.shape
    H = W1.shape[1]
    C = W2.shape[1]

    x_flat = x.astype(jnp.int32).reshape(-1)
    emb_flat = embed.reshape(-1)
    h = _sc_pool(x_flat, emb_flat, B, L, D, V).reshape(B, D)

    out = pl.pallas_call(
        _mlp_body,
        out_shape=jax.ShapeDtypeStruct((B, C), jnp.float32),
    )(h, W1, b1.reshape(1, H), W2, b2.reshape(1, C))
    return out


# speedup vs baseline: 1.2555x; 1.2555x over previous
import functools
import jax
import jax.numpy as jnp
from jax import lax
from jax.experimental import pallas as pl
from jax.experimental.pallas import tpu as pltpu
from jax.experimental.pallas import tpu_sc as plsc

_NC = 2
_NS = 16
_NW = _NC * _NS
_L = 16
_UNROLL = 4


def _sc_pool(x_flat, emb_flat, B, L, D, V):
    spw = B // _NW
    groups = spw // _L
    inv_l = 1.0 / float(L)

    mesh = plsc.VectorSubcoreMesh(
        core_axis_name="c", subcore_axis_name="s",
        num_cores=_NC, num_subcores=_NS,
    )

    @functools.partial(
        pl.kernel,
        out_type=jax.ShapeDtypeStruct((B * D,), jnp.float32),
        mesh=mesh,
        compiler_params=pltpu.CompilerParams(needs_layout_passes=False),
        scratch_types=[
            pltpu.VMEM((spw * L,), jnp.int32),
            pltpu.VMEM((V * D,), jnp.float32),
            pltpu.VMEM((spw * D,), jnp.float32),
        ],
    )
    def pool_kernel(x_hbm, emb_hbm, out_hbm, x_v, emb_v, h_v):
        wid = lax.axis_index("s") * _NC + lax.axis_index("c")
        pltpu.sync_copy(x_hbm.at[pl.ds(wid * spw * L, spw * L)], x_v)
        pltpu.sync_copy(emb_hbm, emb_v)

        lane = lax.iota(jnp.int32, _L)
        row_off = lane * L
        lane_d = lane * D

        def group_body(g, carry):
            x_base = g * (_L * L)

            def tok_body(t, accs):
                l0 = t * _UNROLL
                for u in range(_UNROLL):
                    idxv = plsc.load_gather(x_v, [row_off + (x_base + l0 + u)])
                    idxd = idxv * D
                    accs = tuple(
                        accs[d] + plsc.load_gather(emb_v, [idxd + d])
                        for d in range(D)
                    )
                return accs

            zeros = jnp.zeros((_L,), jnp.float32)
            accs = lax.fori_loop(0, L // _UNROLL, tok_body, (zeros,) * D)
            h_base = g * (_L * D)
            for d in range(D):
                plsc.store_scatter(h_v, [lane_d + (h_base + d)], accs[d] * inv_l)
            return carry

        lax.fori_loop(0, groups, group_body, 0)
        pltpu.sync_copy(h_v, out_hbm.at[pl.ds(wid * spw * D, spw * D)])

    return pool_kernel(x_flat, emb_flat)


def _mlp_body(h_ref, w1_ref, b1_ref, w2_ref, b2_ref, o_ref):
    h = h_ref[...]
    z = jnp.dot(h, w1_ref[...], preferred_element_type=jnp.float32) + b1_ref[...]
    z = jnp.maximum(z, 0.0)
    o_ref[...] = jnp.dot(z, w2_ref[...], preferred_element_type=jnp.float32) + b2_ref[...]


def kernel(x, embed, W1, b1, W2, b2):
    B, L = x.shape
    V, D = embed.shape
    H = W1.shape[1]
    C = W2.shape[1]

    x_flat = x.astype(jnp.int32).reshape(-1)
    emb_flat = embed.reshape(-1)
    h = _sc_pool(x_flat, emb_flat, B, L, D, V).reshape(B, D)

    out = pl.pallas_call(
        _mlp_body,
        out_shape=jax.ShapeDtypeStruct((B, C), jnp.float32),
    )(h, W1, b1.reshape(1, H), W2, b2.reshape(1, C))
    return out


# speedup vs baseline: 1.3195x; 1.0510x over previous
import functools
import jax
import jax.numpy as jnp
from jax import lax
from jax.experimental import pallas as pl
from jax.experimental.pallas import tpu as pltpu
from jax.experimental.pallas import tpu_sc as plsc

_NC = 2
_NS = 16
_NW = _NC * _NS
_L = 16


def _sc_pool(x_flat, emb_flat, B, L, D, V):
    spw = B // _NW
    groups = spw // _L
    inv_l = 1.0 / float(L)

    mesh = plsc.VectorSubcoreMesh(
        core_axis_name="c", subcore_axis_name="s",
        num_cores=_NC, num_subcores=_NS,
    )

    @functools.partial(
        pl.kernel,
        out_type=jax.ShapeDtypeStruct((B * D,), jnp.float32),
        mesh=mesh,
        compiler_params=pltpu.CompilerParams(needs_layout_passes=False),
        scratch_types=[
            pltpu.VMEM((spw * L + _L,), jnp.int32),
            pltpu.VMEM((V * D,), jnp.float32),
            pltpu.VMEM((spw * D,), jnp.float32),
        ],
    )
    def pool_kernel(x_hbm, emb_hbm, out_hbm, x_v, emb_v, h_v):
        wid = lax.axis_index("s") * _NC + lax.axis_index("c")
        pltpu.sync_copy(x_hbm.at[pl.ds(wid * spw * L, spw * L)],
                        x_v.at[pl.ds(0, spw * L)])
        pltpu.sync_copy(emb_hbm, emb_v)

        lane = lax.iota(jnp.int32, _L)
        row_off = lane * L
        lane_d = lane * D
        vmask = V - 1

        def group_body(g, carry):
            x_base = g * (_L * L)

            def tok_body(t, st):
                accs, pend, idxv = st
                accs = tuple(accs[d] + pend[d] for d in range(D))
                idxd = (idxv & vmask) * D
                pend = tuple(
                    plsc.load_gather(emb_v, [idxd + d]) for d in range(D)
                )
                nxt = plsc.load_gather(x_v, [row_off + (x_base + t + 1)])
                return (accs, pend, nxt)

            zeros = (jnp.zeros((_L,), jnp.float32),) * D
            idxv0 = plsc.load_gather(x_v, [row_off + x_base])
            accs, pend, _ = lax.fori_loop(0, L, tok_body, (zeros, zeros, idxv0))
            h_base = g * (_L * D)
            for d in range(D):
                plsc.store_scatter(
                    h_v, [lane_d + (h_base + d)], (accs[d] + pend[d]) * inv_l)
            return carry

        lax.fori_loop(0, groups, group_body, 0)
        pltpu.sync_copy(h_v, out_hbm.at[pl.ds(wid * spw * D, spw * D)])

    return pool_kernel(x_flat, emb_flat)


def _mlp_body(h_ref, w1_ref, b1_ref, w2_ref, b2_ref, o_ref):
    h = h_ref[...]
    z = jnp.dot(h, w1_ref[...], preferred_element_type=jnp.float32) + b1_ref[...]
    z = jnp.maximum(z, 0.0)
    o_ref[...] = jnp.dot(z, w2_ref[...], preferred_element_type=jnp.float32) + b2_ref[...]


def kernel(x, embed, W1, b1, W2, b2):
    B, L = x.shape
    V, D = embed.shape
    H = W1.shape[1]
    C = W2.shape[1]

    x_flat = x.astype(jnp.int32).reshape(-1)
    emb_flat = embed.reshape(-1)
    h = _sc_pool(x_flat, emb_flat, B, L, D, V).reshape(B, D)

    out = pl.pallas_call(
        _mlp_body,
        out_shape=jax.ShapeDtypeStruct((B, C), jnp.float32),
    )(h, W1, b1.reshape(1, H), W2, b2.reshape(1, C))
    return out


# speedup vs baseline: 2.5996x; 1.9701x over previous
import functools
import jax
import jax.numpy as jnp
from jax import lax
from jax.experimental import pallas as pl
from jax.experimental.pallas import tpu as pltpu
from jax.experimental.pallas import tpu_sc as plsc

_NC = 2
_NS = 16
_NW = _NC * _NS
_L = 16


def _sc_pool(x_flat, emb_packed, B, L, D, V):
    spw = B // _NW
    groups = spw // _L
    P = D // 2
    inv_l = 1.0 / float(L)

    mesh = plsc.VectorSubcoreMesh(
        core_axis_name="c", subcore_axis_name="s",
        num_cores=_NC, num_subcores=_NS,
    )

    @functools.partial(
        pl.kernel,
        out_type=jax.ShapeDtypeStruct((B * D,), jnp.float32),
        mesh=mesh,
        compiler_params=pltpu.CompilerParams(needs_layout_passes=False),
        scratch_types=[
            pltpu.VMEM((spw * L + _L,), jnp.int32),
            pltpu.VMEM((V * D // 2,), jnp.int32),
            pltpu.VMEM((spw * D,), jnp.float32),
        ],
    )
    def pool_kernel(x_hbm, emb_hbm, out_hbm, x_v, emb_v, h_v):
        wid = lax.axis_index("s") * _NC + lax.axis_index("c")
        pltpu.sync_copy(x_hbm.at[pl.ds(wid * spw * L, spw * L)],
                        x_v.at[pl.ds(0, spw * L)])
        pltpu.sync_copy(emb_hbm, emb_v)

        lane = lax.iota(jnp.int32, _L)
        row_off = lane * L
        lane_d = lane * D
        vmask = V - 1
        himask = jnp.int32(-65536)

        def group_body(g, carry):
            x_base = g * (_L * L)

            def tok_body(t, st):
                accs, pend, idxv = st
                accs = tuple(
                    (accs[2 * p] + plsc.bitcast(pend[p] << 16, jnp.float32),
                     accs[2 * p + 1] + plsc.bitcast(pend[p] & himask,
                                                    jnp.float32))
                    for p in range(P)
                )
                accs = tuple(a for pair in accs for a in pair)
                idxp = (idxv & vmask) * P
                pend = tuple(
                    plsc.load_gather(emb_v, [idxp + p]) for p in range(P)
                )
                nxt = plsc.load_gather(x_v, [row_off + (x_base + t + 1)])
                return (accs, pend, nxt)

            zeros = (jnp.zeros((_L,), jnp.float32),) * D
            pend0 = (jnp.zeros((_L,), jnp.int32),) * P
            idxv0 = plsc.load_gather(x_v, [row_off + x_base])
            accs, pend, _ = lax.fori_loop(0, L, tok_body, (zeros, pend0, idxv0))
            h_base = g * (_L * D)
            for p in range(P):
                lo = plsc.bitcast(pend[p] << 16, jnp.float32)
                hi = plsc.bitcast(pend[p] & himask, jnp.float32)
                plsc.store_scatter(
                    h_v, [lane_d + (h_base + 2 * p)],
                    (accs[2 * p] + lo) * inv_l)
                plsc.store_scatter(
                    h_v, [lane_d + (h_base + 2 * p + 1)],
                    (accs[2 * p + 1] + hi) * inv_l)
            return carry

        lax.fori_loop(0, groups, group_body, 0)
        pltpu.sync_copy(h_v, out_hbm.at[pl.ds(wid * spw * D, spw * D)])

    return pool_kernel(x_flat, emb_packed)


def _mlp_body(h_ref, w1_ref, b1_ref, w2_ref, b2_ref, o_ref):
    h = h_ref[...]
    z = jnp.dot(h, w1_ref[...], preferred_element_type=jnp.float32) + b1_ref[...]
    z = jnp.maximum(z, 0.0)
    o_ref[...] = jnp.dot(z, w2_ref[...], preferred_element_type=jnp.float32) + b2_ref[...]


def kernel(x, embed, W1, b1, W2, b2):
    B, L = x.shape
    V, D = embed.shape
    H = W1.shape[1]
    C = W2.shape[1]

    x_flat = x.astype(jnp.int32).reshape(-1)
    eb = lax.bitcast_convert_type(
        embed.astype(jnp.bfloat16), jnp.uint16).astype(jnp.uint32)
    emb_packed = lax.bitcast_convert_type(
        eb[:, 0::2] | (eb[:, 1::2] << 16), jnp.int32).reshape(-1)
    h = _sc_pool(x_flat, emb_packed, B, L, D, V).reshape(B, D)

    out = pl.pallas_call(
        _mlp_body,
        out_shape=jax.ShapeDtypeStruct((B, C), jnp.float32),
    )(h, W1, b1.reshape(1, H), W2, b2.reshape(1, C))
    return out
